# baseline (speedup 1.0000x reference)
import functools

import jax
import jax.numpy as jnp
from jax import lax
from jax.experimental import pallas as pl
from jax.experimental.pallas import tpu as pltpu
from jax.experimental.pallas import tpu_sc as plsc

CROP_RATIO = 0.8


@functools.lru_cache(maxsize=None)
def _crop_call(B, L, C, crop_len):
    info = plsc.get_sparse_core_info()
    NC, NS, NL = info.num_cores, info.num_subcores, info.num_lanes
    NW = NC * NS
    assert B % NL == 0
    ROWS = 64
    TCH = ROWS // B
    assert crop_len % TCH == 0
    NCHT = crop_len // TCH
    q, extra = divmod(NCHT, NW)
    NJ = q + (1 if extra else 0)
    NBUF = 7
    PADC = (q * (NW - 1) + min(NW - 1, extra)) + NJ

    mesh = plsc.VectorSubcoreMesh(core_axis_name="c", subcore_axis_name="s")

    @functools.partial(
        pl.kernel,
        mesh=mesh,
        compiler_params=pltpu.CompilerParams(needs_layout_passes=False),
        out_type=jax.ShapeDtypeStruct((crop_len * B, C), jnp.float32),
        scratch_types=[
            pltpu.VMEM((NJ * ROWS,), jnp.int32),
        ]
        + [pltpu.VMEM((ROWS, C), jnp.float32) for _ in range(NBUF)]
        + [pltpu.SemaphoreType.DMA for _ in range(2 * NBUF)],
    )
    def k(x2_hbm, idx_hbm, out_hbm, idx_v, *bufs_sems):
        bufs = bufs_sems[:NBUF]
        rsems = bufs_sems[NBUF : 2 * NBUF]
        wsems = bufs_sems[2 * NBUF :]
        wid = lax.axis_index("c") * NS + lax.axis_index("s")

        c0 = q * wid + jnp.minimum(wid, extra)
        pltpu.sync_copy(
            idx_hbm.at[pl.ds(pl.multiple_of(c0 * ROWS, ROWS), NJ * ROWS)], idx_v
        )

        def tc_of(j):
            return jnp.minimum(c0 + j, NCHT - 1)

        def rd(j):
            return pltpu.make_async_copy(
                x2_hbm.at[idx_v.at[pl.ds(j * ROWS, ROWS)]],
                bufs[j % NBUF],
                rsems[j % NBUF],
            )

        def wr(j):
            return pltpu.make_async_copy(
                bufs[j % NBUF],
                out_hbm.at[pl.ds(pl.multiple_of(tc_of(j) * ROWS, ROWS), ROWS), :],
                wsems[j % NBUF],
            )

        ahead = NBUF - 1
        for j in range(min(ahead, NJ)):
            rd(j).start()
        for j in range(NJ):
            rd(j).wait()
            wr(j).start()
            nxt = j + ahead
            if nxt < NJ:
                if nxt - NBUF >= 0:
                    wr(nxt - NBUF).wait()
                rd(nxt).start()
        for j in range(max(0, NJ - NBUF), NJ):
            wr(j).wait()

    return k, PADC * ROWS, TCH


def kernel(x):
    B, L, C = x.shape
    crop_len = int(L * CROP_RATIO)
    call, pad_rows, tch = _crop_call(B, L, C, crop_len)
    with jax.ensure_compile_time_eval():
        start = jax.random.randint(
            jax.random.key(1), (B,), 0, L - crop_len + 1
        ).astype(jnp.int32)
        t = jnp.arange(pad_rows // B, dtype=jnp.int32)
        t = jnp.where(t >= crop_len, t - tch, t)
        g = jnp.arange(B, dtype=jnp.int32) * L + start
        idx = jnp.asarray((g[None, :] + t[:, None]).reshape(-1))
    out2 = call(x.reshape(B * L, C), idx)
    return out2.reshape(crop_len, B, C).transpose(1, 0, 2)
